# baseline (speedup 1.0000x reference)
import functools

import jax
import jax.numpy as jnp
from jax import lax
from jax.experimental import pallas as pl
from jax.experimental.pallas import tpu as pltpu
from jax.experimental.pallas import tpu_sc as plsc

N, D, M, C, K = 2048, 1024, 512, 64, 16
CK = C * K
NT = 256
GRID = N // NT
NW = 32
NPW = N // NW
MV = M // 16


def _lut_norms_body(p_ref, b_ref, lut_ref, norms_ref):
    p = p_ref[...]
    lut_ref[...] = lax.dot_general(
        p, b_ref[...], (((1,), (0,)), ((), ())),
        preferred_element_type=jnp.float32,
        precision=lax.Precision.DEFAULT)
    norms_ref[...] = jnp.sum(p * p, axis=1, keepdims=True)


def _encode_body(p_ref, at_ref, norms_ref, codes_ref):
    scores = lax.dot_general(
        p_ref[...], at_ref[...], (((1,), (0,)), ((), ())),
        preferred_element_type=jnp.float32,
        precision=lax.Precision.DEFAULT)
    fit = 2.0 * scores - norms_ref[...]
    fit3 = fit.reshape(C, K, NT)
    maxv = jnp.max(fit3, axis=1, keepdims=True)
    kio = lax.broadcasted_iota(jnp.int32, (C, K, NT), 1)
    ksel = jnp.min(jnp.where(fit3 == maxv, kio, K), axis=1)
    cio = lax.broadcasted_iota(jnp.int32, (C, NT), 0)
    codes_ref[...] = ksel + K * cio


def _agg_body(idx_hbm, lut_hbm, out_hbm, idx_v, buf_v, out_v, sem):
    wid = lax.axis_index("s") * 2 + lax.axis_index("c")
    base = wid * NPW
    pltpu.sync_copy(idx_hbm.at[pl.ds(base, NPW)], idx_v)

    def row_body(r, carry):
        pltpu.async_copy(lut_hbm.at[idx_v.at[r]], buf_v, sem).wait()

        def acc_body(t, acc):
            return tuple(acc[j] + buf_v[t, pl.ds(j * 16, 16)] for j in range(MV))

        acc0 = tuple(jnp.zeros((16,), jnp.float32) for _ in range(MV))
        acc = lax.fori_loop(0, C, acc_body, acc0)
        for j in range(MV):
            out_v[r, pl.ds(j * 16, 16)] = acc[j]
        return carry

    lax.fori_loop(0, NPW, row_body, 0)
    pltpu.sync_copy(out_v, out_hbm.at[pl.ds(base, NPW)])


@jax.jit
def kernel(A, B, prototypes):
    P = prototypes.reshape(CK, D)
    lut, norms = pl.pallas_call(
        _lut_norms_body,
        out_shape=(jax.ShapeDtypeStruct((CK, M), jnp.float32),
                   jax.ShapeDtypeStruct((CK, 1), jnp.float32)),
    )(P, B)
    A_T = A.T
    codes_T = pl.pallas_call(
        _encode_body,
        grid=(GRID,),
        in_specs=[
            pl.BlockSpec((CK, D), lambda i: (0, 0)),
            pl.BlockSpec((D, NT), lambda i: (0, i)),
            pl.BlockSpec((CK, 1), lambda i: (0, 0)),
        ],
        out_specs=pl.BlockSpec((C, NT), lambda i: (0, i)),
        out_shape=jax.ShapeDtypeStruct((C, N), jnp.int32),
    )(P, A_T, norms)
    flat_idx = codes_T.T

    agg = pl.kernel(
        _agg_body,
        out_type=jax.ShapeDtypeStruct((N, M), jnp.float32),
        mesh=plsc.VectorSubcoreMesh(core_axis_name="c", subcore_axis_name="s"),
        scratch_types=[
            pltpu.VMEM((NPW, C), jnp.int32),
            pltpu.VMEM((C, M), jnp.float32),
            pltpu.VMEM((NPW, M), jnp.float32),
            pltpu.SemaphoreType.DMA,
        ],
    )
    return agg(flat_idx, lut)
